# Initial kernel scaffold; baseline (speedup 1.0000x reference)
import jax
import jax.numpy as jnp
from jax.experimental import pallas as pl


def kernel(embeddings, edge_index, edge_weight, normalized_A_values, D_values):
    raise NotImplementedError("write your pallas kernel here")


# speedup vs baseline: 3.0710x; 3.0710x over previous
import dataclasses
import functools

import jax
import jax.numpy as jnp
from jax import lax
from jax.experimental import pallas as pl
from jax.experimental.pallas import tpu as pltpu
from jax.experimental.pallas import tpu_sc as plsc

N = 10000
E = 320000
K = 128

NC = 2
NS = 16
NW = NC * NS
EPW = E // NW
C = 80
NCHUNK = EPW // C
LANES = 16


def _sc_compiler_params():
    cp = pltpu.CompilerParams()
    if "needs_layout_passes" in pltpu.CompilerParams.__dataclass_fields__:
        cp = dataclasses.replace(cp, needs_layout_passes=False)
    return cp


def _sc_term1(src, dst, a_vals, emb):
    mesh = plsc.VectorSubcoreMesh(core_axis_name="c", subcore_axis_name="s")

    @functools.partial(
        pl.kernel,
        mesh=mesh,
        compiler_params=_sc_compiler_params(),
        out_type=jax.ShapeDtypeStruct((NW, LANES), jnp.float32),
        scratch_types=[
            pltpu.VMEM((C,), jnp.int32),
            pltpu.VMEM((C,), jnp.int32),
            pltpu.VMEM((C,), jnp.float32),
            pltpu.VMEM((C, K), jnp.float32),
            pltpu.VMEM((C, K), jnp.float32),
            pltpu.VMEM((LANES,), jnp.float32),
            pltpu.SemaphoreType.DMA,
        ],
    )
    def k(src_hbm, dst_hbm, a_hbm, emb_hbm, out_hbm,
          sidx, didx, av, srows, drows, accv, sem):
        wid = lax.axis_index("s") * NC + lax.axis_index("c")
        base = wid * EPW
        accv[...] = jnp.zeros((LANES,), jnp.float32)

        @pl.loop(0, NCHUNK)
        def _chunk(ci):
            off = base + ci * C
            pltpu.sync_copy(src_hbm.at[pl.ds(off, C)], sidx)
            pltpu.sync_copy(dst_hbm.at[pl.ds(off, C)], didx)
            pltpu.sync_copy(a_hbm.at[pl.ds(off, C)], av)
            pltpu.async_copy(emb_hbm.at[sidx], srows, sem).wait()
            pltpu.async_copy(emb_hbm.at[didx], drows, sem).wait()

            def edge_body(e, acc):
                t = srows[e, pl.ds(0, LANES)] * drows[e, pl.ds(0, LANES)]
                for j in range(1, K // LANES):
                    t = t + (srows[e, pl.ds(j * LANES, LANES)]
                             * drows[e, pl.ds(j * LANES, LANES)])
                ab = plsc.load_gather(av, [jnp.full((LANES,), e, jnp.int32)])
                return acc + ab * t

            acc = lax.fori_loop(0, C, edge_body,
                                jnp.zeros((LANES,), jnp.float32))
            accv[...] = accv[...] + acc

        pltpu.sync_copy(accv, out_hbm.at[wid])

    return k(src, dst, a_vals, emb)


_BR = 1000


def _gram_body(e_ref, d_ref, ptp_ref, t2_ref):
    @pl.when(pl.program_id(0) == 0)
    def _():
        ptp_ref[...] = jnp.zeros((K, K), jnp.float32)
        t2_ref[0, 0] = 0.0

    blk = e_ref[...]
    ptp_ref[...] += lax.dot_general(blk, blk, (((0,), (0,)), ((), ())),
                                    preferred_element_type=jnp.float32)
    rs = jnp.sum(blk * blk, axis=1)
    t2_ref[0, 0] += jnp.sum(d_ref[...][:, 0] * rs)


def _tc_gram(emb, d_col):
    return pl.pallas_call(
        _gram_body,
        grid=(N // _BR,),
        in_specs=[
            pl.BlockSpec((_BR, K), lambda i: (i, 0)),
            pl.BlockSpec((_BR, 1), lambda i: (i, 0)),
        ],
        out_specs=[
            pl.BlockSpec((K, K), lambda i: (0, 0)),
            pl.BlockSpec(memory_space=pltpu.SMEM),
        ],
        out_shape=[
            jax.ShapeDtypeStruct((K, K), jnp.float32),
            jax.ShapeDtypeStruct((1, 1), jnp.float32),
        ],
    )(emb, d_col)


def _combine_body(ptp_ref, part_ref, t2_ref, out_ref):
    ptp = ptp_ref[...]
    term1 = jnp.sum(part_ref[...])
    term2 = t2_ref[0, 0]
    n = jnp.sqrt(jnp.sum(ptp * ptp))
    row = lax.broadcasted_iota(jnp.int32, (K, K), 0)
    col = lax.broadcasted_iota(jnp.int32, (K, K), 1)
    eye = jnp.where(row == col, jnp.float32(1.0), jnp.float32(0.0))
    m = ptp / n - eye / jnp.sqrt(jnp.float32(K))
    penalty = jnp.sqrt(jnp.sum(m * m))
    out_ref[0, 0] = -(term1 / term2) + penalty


def _tc_combine(ptp, partials, t2):
    return pl.pallas_call(
        _combine_body,
        in_specs=[
            pl.BlockSpec((K, K), lambda: (0, 0)),
            pl.BlockSpec((NW, LANES), lambda: (0, 0)),
            pl.BlockSpec(memory_space=pltpu.SMEM),
        ],
        out_specs=pl.BlockSpec(memory_space=pltpu.SMEM),
        out_shape=jax.ShapeDtypeStruct((1, 1), jnp.float32),
    )(ptp, partials, t2)


def kernel(embeddings, edge_index, edge_weight, normalized_A_values, D_values):
    del edge_weight
    src = edge_index[0].astype(jnp.int32)
    dst = edge_index[1].astype(jnp.int32)
    partials = _sc_term1(src, dst, normalized_A_values, embeddings)
    ptp, t2 = _tc_gram(embeddings, D_values.reshape(N, 1))
    out = _tc_combine(ptp, partials, t2)
    return out[0, 0]


# speedup vs baseline: 9.0307x; 2.9407x over previous
import dataclasses
import functools

import jax
import jax.numpy as jnp
from jax import lax
from jax.experimental import pallas as pl
from jax.experimental.pallas import tpu as pltpu
from jax.experimental.pallas import tpu_sc as plsc

N = 10000
E = 320000
K = 128

NC = 2
NS = 16
NW = NC * NS
EPW = E // NW
C = 125
NCHUNK = EPW // C
LANES = 16


def _sc_compiler_params():
    cp = pltpu.CompilerParams()
    if "needs_layout_passes" in pltpu.CompilerParams.__dataclass_fields__:
        cp = dataclasses.replace(cp, needs_layout_passes=False)
    return cp


_UNROLL = 5


def _sc_term1(src3, dst3, a3, emb):
    mesh = plsc.VectorSubcoreMesh(core_axis_name="c", subcore_axis_name="s")

    @functools.partial(
        pl.kernel,
        mesh=mesh,
        compiler_params=_sc_compiler_params(),
        out_type=jax.ShapeDtypeStruct((NW, LANES), jnp.float32),
        scratch_types=[
            pltpu.VMEM((NCHUNK, C), jnp.int32),
            pltpu.VMEM((NCHUNK, C), jnp.int32),
            pltpu.VMEM((NCHUNK, C), jnp.float32),
            pltpu.VMEM((C, K), jnp.float32),
            pltpu.VMEM((C, K), jnp.float32),
            pltpu.VMEM((C, K), jnp.float32),
            pltpu.VMEM((C, K), jnp.float32),
            pltpu.VMEM((LANES,), jnp.float32),
            pltpu.SemaphoreType.DMA,
            pltpu.SemaphoreType.DMA,
            pltpu.SemaphoreType.DMA,
            pltpu.SemaphoreType.DMA,
        ],
    )
    def k(src_hbm, dst_hbm, a_hbm, emb_hbm, out_hbm,
          sidx_all, didx_all, av_all, sr0, dr0, sr1, dr1, accv,
          ss0, sd0, ss1, sd1):
        wid = lax.axis_index("s") * NC + lax.axis_index("c")
        pltpu.sync_copy(src_hbm.at[wid], sidx_all)
        pltpu.sync_copy(dst_hbm.at[wid], didx_all)
        pltpu.sync_copy(a_hbm.at[wid], av_all)
        accv[...] = jnp.zeros((LANES,), jnp.float32)

        def issue(ci, sbuf, dbuf, ssem, dsem):
            pltpu.async_copy(emb_hbm.at[sidx_all.at[ci]], sbuf, ssem)
            pltpu.async_copy(emb_hbm.at[didx_all.at[ci]], dbuf, dsem)

        def wait(ci, sbuf, dbuf, ssem, dsem):
            pltpu.make_async_copy(emb_hbm.at[sidx_all.at[ci]], sbuf, ssem).wait()
            pltpu.make_async_copy(emb_hbm.at[didx_all.at[ci]], dbuf, dsem).wait()

        def compute(ci, sbuf, dbuf):
            def edge_group(g, acc):
                e0 = g * _UNROLL
                for u in range(_UNROLL):
                    e = e0 + u
                    t = sbuf[e, pl.ds(0, LANES)] * dbuf[e, pl.ds(0, LANES)]
                    for j in range(1, K // LANES):
                        t = t + (sbuf[e, pl.ds(j * LANES, LANES)]
                                 * dbuf[e, pl.ds(j * LANES, LANES)])
                    ab = plsc.load_gather(
                        av_all, [jnp.full((LANES,), ci, jnp.int32),
                                 jnp.full((LANES,), e, jnp.int32)])
                    acc = acc + ab * t
                return acc

            acc = lax.fori_loop(0, C // _UNROLL, edge_group,
                                jnp.zeros((LANES,), jnp.float32))
            accv[...] = accv[...] + acc

        issue(0, sr0, dr0, ss0, sd0)

        @pl.loop(0, NCHUNK, step=2)
        def _chunk(ci):
            issue(ci + 1, sr1, dr1, ss1, sd1)
            wait(ci, sr0, dr0, ss0, sd0)
            compute(ci, sr0, dr0)

            @pl.when(ci + 2 < NCHUNK)
            def _():
                issue(ci + 2, sr0, dr0, ss0, sd0)

            wait(ci + 1, sr1, dr1, ss1, sd1)
            compute(ci + 1, sr1, dr1)

        pltpu.sync_copy(accv, out_hbm.at[wid])

    return k(src3, dst3, a3, emb)


_BR = 1000


def _gram_body(e_ref, d_ref, ptp_ref, t2_ref):
    @pl.when(pl.program_id(0) == 0)
    def _():
        ptp_ref[...] = jnp.zeros((K, K), jnp.float32)
        t2_ref[0, 0] = 0.0

    blk = e_ref[...]
    ptp_ref[...] += lax.dot_general(blk, blk, (((0,), (0,)), ((), ())),
                                    preferred_element_type=jnp.float32)
    rs = jnp.sum(blk * blk, axis=1)
    t2_ref[0, 0] += jnp.sum(d_ref[...][:, 0] * rs)


def _tc_gram(emb, d_col):
    return pl.pallas_call(
        _gram_body,
        grid=(N // _BR,),
        in_specs=[
            pl.BlockSpec((_BR, K), lambda i: (i, 0)),
            pl.BlockSpec((_BR, 1), lambda i: (i, 0)),
        ],
        out_specs=[
            pl.BlockSpec((K, K), lambda i: (0, 0)),
            pl.BlockSpec(memory_space=pltpu.SMEM),
        ],
        out_shape=[
            jax.ShapeDtypeStruct((K, K), jnp.float32),
            jax.ShapeDtypeStruct((1, 1), jnp.float32),
        ],
    )(emb, d_col)


def _combine_body(ptp_ref, part_ref, t2_ref, out_ref):
    ptp = ptp_ref[...]
    term1 = jnp.sum(part_ref[...])
    term2 = t2_ref[0, 0]
    n = jnp.sqrt(jnp.sum(ptp * ptp))
    row = lax.broadcasted_iota(jnp.int32, (K, K), 0)
    col = lax.broadcasted_iota(jnp.int32, (K, K), 1)
    eye = jnp.where(row == col, jnp.float32(1.0), jnp.float32(0.0))
    m = ptp / n - eye / jnp.sqrt(jnp.float32(K))
    penalty = jnp.sqrt(jnp.sum(m * m))
    out_ref[0, 0] = -(term1 / term2) + penalty


def _tc_combine(ptp, partials, t2):
    return pl.pallas_call(
        _combine_body,
        in_specs=[
            pl.BlockSpec((K, K), lambda: (0, 0)),
            pl.BlockSpec((NW, LANES), lambda: (0, 0)),
            pl.BlockSpec(memory_space=pltpu.SMEM),
        ],
        out_specs=pl.BlockSpec(memory_space=pltpu.SMEM),
        out_shape=jax.ShapeDtypeStruct((1, 1), jnp.float32),
    )(ptp, partials, t2)


def kernel(embeddings, edge_index, edge_weight, normalized_A_values, D_values):
    del edge_weight
    src = edge_index[0].astype(jnp.int32).reshape(NW, NCHUNK, C)
    dst = edge_index[1].astype(jnp.int32).reshape(NW, NCHUNK, C)
    a3 = normalized_A_values.reshape(NW, NCHUNK, C)
    partials = _sc_term1(src, dst, a3, embeddings)
    ptp, t2 = _tc_gram(embeddings, D_values.reshape(N, 1))
    out = _tc_combine(ptp, partials, t2)
    return out[0, 0]


# speedup vs baseline: 10.0791x; 1.1161x over previous
import dataclasses
import functools

import jax
import jax.numpy as jnp
from jax import lax
from jax.experimental import pallas as pl
from jax.experimental.pallas import tpu as pltpu
from jax.experimental.pallas import tpu_sc as plsc

N = 10000
E = 320000
K = 128

NC = 2
NS = 16
NW = NC * NS
EPW = E // NW
C = 125
NCHUNK = EPW // C
LANES = 16


def _sc_compiler_params():
    cp = pltpu.CompilerParams()
    if "needs_layout_passes" in pltpu.CompilerParams.__dataclass_fields__:
        cp = dataclasses.replace(cp, needs_layout_passes=False)
    if "use_tc_tiling_on_sc" in pltpu.CompilerParams.__dataclass_fields__:
        cp = dataclasses.replace(cp, use_tc_tiling_on_sc=False)
    return cp


_UNROLL = 5


def _sc_term1(src3, dst3, a3, emb):
    mesh = plsc.VectorSubcoreMesh(core_axis_name="c", subcore_axis_name="s")

    @functools.partial(
        pl.kernel,
        mesh=mesh,
        compiler_params=_sc_compiler_params(),
        out_type=jax.ShapeDtypeStruct((NW, LANES), jnp.float32),
        scratch_types=[
            pltpu.VMEM((NCHUNK, C), jnp.int32),
            pltpu.VMEM((NCHUNK, C), jnp.int32),
            pltpu.VMEM((NCHUNK, C), jnp.float32),
            pltpu.VMEM((C, K // 2), jnp.int32),
            pltpu.VMEM((C, K // 2), jnp.int32),
            pltpu.VMEM((C, K // 2), jnp.int32),
            pltpu.VMEM((C, K // 2), jnp.int32),
            pltpu.VMEM((LANES,), jnp.float32),
            pltpu.SemaphoreType.DMA,
            pltpu.SemaphoreType.DMA,
            pltpu.SemaphoreType.DMA,
            pltpu.SemaphoreType.DMA,
        ],
    )
    def k(src_hbm, dst_hbm, a_hbm, emb_hbm, out_hbm,
          sidx_all, didx_all, av_all, sr0, dr0, sr1, dr1, accv,
          ss0, sd0, ss1, sd1):
        wid = lax.axis_index("s") * NC + lax.axis_index("c")
        pltpu.sync_copy(src_hbm.at[wid], sidx_all)
        pltpu.sync_copy(dst_hbm.at[wid], didx_all)
        pltpu.sync_copy(a_hbm.at[wid], av_all)
        accv[...] = jnp.zeros((LANES,), jnp.float32)

        def issue(ci, sbuf, dbuf, ssem, dsem):
            pltpu.async_copy(emb_hbm.at[sidx_all.at[ci]], sbuf, ssem)
            pltpu.async_copy(emb_hbm.at[didx_all.at[ci]], dbuf, dsem)

        def wait(ci, sbuf, dbuf, ssem, dsem):
            pltpu.make_async_copy(emb_hbm.at[sidx_all.at[ci]], sbuf, ssem).wait()
            pltpu.make_async_copy(emb_hbm.at[didx_all.at[ci]], dbuf, dsem).wait()

        def compute(ci, sbuf, dbuf):
            def edge_group(g, acc):
                e0 = g * _UNROLL
                for u in range(_UNROLL):
                    e = e0 + u
                    t = jnp.zeros((LANES,), jnp.float32)
                    for j in range(K // (2 * LANES)):
                        sv = plsc.bitcast(
                            sbuf[e, pl.ds(j * LANES, LANES)], jnp.bfloat16)
                        dv = plsc.bitcast(
                            dbuf[e, pl.ds(j * LANES, LANES)], jnp.bfloat16)
                        p0, p1 = plsc.unpack(
                            sv * dv, format=plsc.PackFormat.INTERLEAVED)
                        t = t + p0 + p1
                    ab = plsc.load_gather(
                        av_all, [jnp.full((LANES,), ci, jnp.int32),
                                 jnp.full((LANES,), e, jnp.int32)])
                    acc = acc + ab * t
                return acc

            acc = lax.fori_loop(0, C // _UNROLL, edge_group,
                                jnp.zeros((LANES,), jnp.float32))
            accv[...] = accv[...] + acc

        issue(0, sr0, dr0, ss0, sd0)

        @pl.loop(0, NCHUNK, step=2)
        def _chunk(ci):
            issue(ci + 1, sr1, dr1, ss1, sd1)
            wait(ci, sr0, dr0, ss0, sd0)
            compute(ci, sr0, dr0)

            @pl.when(ci + 2 < NCHUNK)
            def _():
                issue(ci + 2, sr0, dr0, ss0, sd0)

            wait(ci + 1, sr1, dr1, ss1, sd1)
            compute(ci + 1, sr1, dr1)

        pltpu.sync_copy(accv, out_hbm.at[wid])

    return k(src3, dst3, a3, emb)


_BR = 1000


def _gram_body(e_ref, d_ref, ptp_ref, t2_ref):
    @pl.when(pl.program_id(0) == 0)
    def _():
        ptp_ref[...] = jnp.zeros((K, K), jnp.float32)
        t2_ref[0, 0] = 0.0

    blk = e_ref[...]
    ptp_ref[...] += lax.dot_general(blk, blk, (((0,), (0,)), ((), ())),
                                    preferred_element_type=jnp.float32)
    rs = jnp.sum(blk * blk, axis=1)
    t2_ref[0, 0] += jnp.sum(d_ref[...][:, 0] * rs)


def _tc_gram(emb, d_col):
    return pl.pallas_call(
        _gram_body,
        grid=(N // _BR,),
        in_specs=[
            pl.BlockSpec((_BR, K), lambda i: (i, 0)),
            pl.BlockSpec((_BR, 1), lambda i: (i, 0)),
        ],
        out_specs=[
            pl.BlockSpec((K, K), lambda i: (0, 0)),
            pl.BlockSpec(memory_space=pltpu.SMEM),
        ],
        out_shape=[
            jax.ShapeDtypeStruct((K, K), jnp.float32),
            jax.ShapeDtypeStruct((1, 1), jnp.float32),
        ],
    )(emb, d_col)


def _combine_body(ptp_ref, part_ref, t2_ref, out_ref):
    ptp = ptp_ref[...]
    term1 = jnp.sum(part_ref[...])
    term2 = t2_ref[0, 0]
    n = jnp.sqrt(jnp.sum(ptp * ptp))
    row = lax.broadcasted_iota(jnp.int32, (K, K), 0)
    col = lax.broadcasted_iota(jnp.int32, (K, K), 1)
    eye = jnp.where(row == col, jnp.float32(1.0), jnp.float32(0.0))
    m = ptp / n - eye / jnp.sqrt(jnp.float32(K))
    penalty = jnp.sqrt(jnp.sum(m * m))
    out_ref[0, 0] = -(term1 / term2) + penalty


def _tc_combine(ptp, partials, t2):
    return pl.pallas_call(
        _combine_body,
        in_specs=[
            pl.BlockSpec((K, K), lambda: (0, 0)),
            pl.BlockSpec((NW, LANES), lambda: (0, 0)),
            pl.BlockSpec(memory_space=pltpu.SMEM),
        ],
        out_specs=pl.BlockSpec(memory_space=pltpu.SMEM),
        out_shape=jax.ShapeDtypeStruct((1, 1), jnp.float32),
    )(ptp, partials, t2)


def kernel(embeddings, edge_index, edge_weight, normalized_A_values, D_values):
    del edge_weight
    src = edge_index[0].astype(jnp.int32).reshape(NW, NCHUNK, C)
    dst = edge_index[1].astype(jnp.int32).reshape(NW, NCHUNK, C)
    a3 = normalized_A_values.reshape(NW, NCHUNK, C)
    emb_packed = lax.bitcast_convert_type(
        embeddings.astype(jnp.bfloat16).reshape(N, K // 2, 2), jnp.int32)
    partials = _sc_term1(src, dst, a3, emb_packed)
    ptp, t2 = _tc_gram(embeddings, D_values.reshape(N, 1))
    out = _tc_combine(ptp, partials, t2)
    return out[0, 0]


# speedup vs baseline: 10.6175x; 1.0534x over previous
import dataclasses
import functools

import jax
import jax.numpy as jnp
from jax import lax
from jax.experimental import pallas as pl
from jax.experimental.pallas import tpu as pltpu
from jax.experimental.pallas import tpu_sc as plsc

N = 10000
E = 320000
K = 128

NC = 2
NS = 16
NW = NC * NS
EPW = E // NW
C = 125
NCHUNK = EPW // C
LANES = 16


def _sc_compiler_params():
    cp = pltpu.CompilerParams()
    if "needs_layout_passes" in pltpu.CompilerParams.__dataclass_fields__:
        cp = dataclasses.replace(cp, needs_layout_passes=False)
    if "use_tc_tiling_on_sc" in pltpu.CompilerParams.__dataclass_fields__:
        cp = dataclasses.replace(cp, use_tc_tiling_on_sc=False)
    return cp


_UNROLL = 5


def _sc_term1(src3, dst3, a3, emb):
    mesh = plsc.VectorSubcoreMesh(core_axis_name="c", subcore_axis_name="s")

    @functools.partial(
        pl.kernel,
        mesh=mesh,
        compiler_params=_sc_compiler_params(),
        out_type=jax.ShapeDtypeStruct((NW, LANES), jnp.float32),
        scratch_types=[
            pltpu.VMEM((NCHUNK, C), jnp.int32),
            pltpu.VMEM((NCHUNK, C), jnp.int32),
            pltpu.VMEM((NCHUNK, C), jnp.float32),
            pltpu.VMEM((C, K // 2), jnp.int32),
            pltpu.VMEM((C, K // 2), jnp.int32),
            pltpu.VMEM((C, K // 2), jnp.int32),
            pltpu.VMEM((C, K // 2), jnp.int32),
            pltpu.VMEM((C, K // 2), jnp.int32),
            pltpu.VMEM((C, K // 2), jnp.int32),
            pltpu.VMEM((C, K // 2), jnp.int32),
            pltpu.VMEM((C, K // 2), jnp.int32),
            pltpu.VMEM((LANES,), jnp.float32),
            pltpu.SemaphoreType.DMA,
            pltpu.SemaphoreType.DMA,
            pltpu.SemaphoreType.DMA,
            pltpu.SemaphoreType.DMA,
            pltpu.SemaphoreType.DMA,
            pltpu.SemaphoreType.DMA,
            pltpu.SemaphoreType.DMA,
            pltpu.SemaphoreType.DMA,
        ],
    )
    def k(src_hbm, dst_hbm, a_hbm, emb_hbm, out_hbm,
          sidx_all, didx_all, av_all,
          sr0, dr0, sr1, dr1, sr2, dr2, sr3, dr3, accv,
          ss0, sd0, ss1, sd1, ss2, sd2, ss3, sd3):
        wid = lax.axis_index("s") * NC + lax.axis_index("c")
        pltpu.sync_copy(src_hbm.at[wid], sidx_all)
        pltpu.sync_copy(dst_hbm.at[wid], didx_all)
        pltpu.sync_copy(a_hbm.at[wid], av_all)
        accv[...] = jnp.zeros((LANES,), jnp.float32)

        bufs = ((sr0, dr0, ss0, sd0), (sr1, dr1, ss1, sd1),
                (sr2, dr2, ss2, sd2), (sr3, dr3, ss3, sd3))
        DEPTH = len(bufs)

        def issue(ci, sbuf, dbuf, ssem, dsem):
            pltpu.async_copy(emb_hbm.at[sidx_all.at[ci]], sbuf, ssem)
            pltpu.async_copy(emb_hbm.at[didx_all.at[ci]], dbuf, dsem)

        def wait(ci, sbuf, dbuf, ssem, dsem):
            pltpu.make_async_copy(emb_hbm.at[sidx_all.at[ci]], sbuf, ssem).wait()
            pltpu.make_async_copy(emb_hbm.at[didx_all.at[ci]], dbuf, dsem).wait()

        def compute(ci, sbuf, dbuf):
            def edge_group(g, acc):
                e0 = g * _UNROLL
                for u in range(_UNROLL):
                    e = e0 + u
                    t = jnp.zeros((LANES,), jnp.float32)
                    for j in range(K // (2 * LANES)):
                        sv = plsc.bitcast(
                            sbuf[e, pl.ds(j * LANES, LANES)], jnp.bfloat16)
                        dv = plsc.bitcast(
                            dbuf[e, pl.ds(j * LANES, LANES)], jnp.bfloat16)
                        p0, p1 = plsc.unpack(
                            sv * dv, format=plsc.PackFormat.INTERLEAVED)
                        t = t + p0 + p1
                    ab = plsc.load_gather(
                        av_all, [jnp.full((LANES,), ci, jnp.int32),
                                 jnp.full((LANES,), e, jnp.int32)])
                    acc = acc + ab * t
                return acc

            acc = lax.fori_loop(0, C // _UNROLL, edge_group,
                                jnp.zeros((LANES,), jnp.float32))
            accv[...] = accv[...] + acc

        for b in range(DEPTH):
            issue(b, *bufs[b])

        @pl.loop(0, NCHUNK, step=DEPTH)
        def _chunk(ci):
            for b in range(DEPTH):
                sbuf, dbuf, ssem, dsem = bufs[b]
                wait(ci + b, sbuf, dbuf, ssem, dsem)
                compute(ci + b, sbuf, dbuf)

                @pl.when(ci + b + DEPTH < NCHUNK)
                def _():
                    issue(ci + b + DEPTH, sbuf, dbuf, ssem, dsem)

        pltpu.sync_copy(accv, out_hbm.at[wid])

    return k(src3, dst3, a3, emb)


_BR = 1000


def _gram_body(e_ref, d_ref, ptp_ref, t2_ref):
    @pl.when(pl.program_id(0) == 0)
    def _():
        ptp_ref[...] = jnp.zeros((K, K), jnp.float32)
        t2_ref[0, 0] = 0.0

    blk = e_ref[...]
    ptp_ref[...] += lax.dot_general(blk, blk, (((0,), (0,)), ((), ())),
                                    preferred_element_type=jnp.float32)
    rs = jnp.sum(blk * blk, axis=1)
    t2_ref[0, 0] += jnp.sum(d_ref[...][:, 0] * rs)


def _tc_gram(emb, d_col):
    return pl.pallas_call(
        _gram_body,
        grid=(N // _BR,),
        in_specs=[
            pl.BlockSpec((_BR, K), lambda i: (i, 0)),
            pl.BlockSpec((_BR, 1), lambda i: (i, 0)),
        ],
        out_specs=[
            pl.BlockSpec((K, K), lambda i: (0, 0)),
            pl.BlockSpec(memory_space=pltpu.SMEM),
        ],
        out_shape=[
            jax.ShapeDtypeStruct((K, K), jnp.float32),
            jax.ShapeDtypeStruct((1, 1), jnp.float32),
        ],
    )(emb, d_col)


def _combine_body(ptp_ref, part_ref, t2_ref, out_ref):
    ptp = ptp_ref[...]
    term1 = jnp.sum(part_ref[...])
    term2 = t2_ref[0, 0]
    n = jnp.sqrt(jnp.sum(ptp * ptp))
    row = lax.broadcasted_iota(jnp.int32, (K, K), 0)
    col = lax.broadcasted_iota(jnp.int32, (K, K), 1)
    eye = jnp.where(row == col, jnp.float32(1.0), jnp.float32(0.0))
    m = ptp / n - eye / jnp.sqrt(jnp.float32(K))
    penalty = jnp.sqrt(jnp.sum(m * m))
    out_ref[0, 0] = -(term1 / term2) + penalty


def _tc_combine(ptp, partials, t2):
    return pl.pallas_call(
        _combine_body,
        in_specs=[
            pl.BlockSpec((K, K), lambda: (0, 0)),
            pl.BlockSpec((NW, LANES), lambda: (0, 0)),
            pl.BlockSpec(memory_space=pltpu.SMEM),
        ],
        out_specs=pl.BlockSpec(memory_space=pltpu.SMEM),
        out_shape=jax.ShapeDtypeStruct((1, 1), jnp.float32),
    )(ptp, partials, t2)


def kernel(embeddings, edge_index, edge_weight, normalized_A_values, D_values):
    del edge_weight
    src = edge_index[0].astype(jnp.int32).reshape(NW, NCHUNK, C)
    dst = edge_index[1].astype(jnp.int32).reshape(NW, NCHUNK, C)
    a3 = normalized_A_values.reshape(NW, NCHUNK, C)
    emb_packed = lax.bitcast_convert_type(
        embeddings.astype(jnp.bfloat16).reshape(N, K // 2, 2), jnp.int32)
    partials = _sc_term1(src, dst, a3, emb_packed)
    ptp, t2 = _tc_gram(embeddings, D_values.reshape(N, 1))
    out = _tc_combine(ptp, partials, t2)
    return out[0, 0]


# speedup vs baseline: 11.3747x; 1.0713x over previous
import dataclasses
import functools

import jax
import jax.numpy as jnp
from jax import lax
from jax.experimental import pallas as pl
from jax.experimental.pallas import tpu as pltpu
from jax.experimental.pallas import tpu_sc as plsc

N = 10000
E = 320000
K = 128

NC = 2
NS = 16
NW = NC * NS
EPW = E // NW
C = 125
NCHUNK = EPW // C
LANES = 16


def _sc_compiler_params():
    cp = pltpu.CompilerParams()
    if "needs_layout_passes" in pltpu.CompilerParams.__dataclass_fields__:
        cp = dataclasses.replace(cp, needs_layout_passes=False)
    if "use_tc_tiling_on_sc" in pltpu.CompilerParams.__dataclass_fields__:
        cp = dataclasses.replace(cp, use_tc_tiling_on_sc=False)
    return cp


_UNROLL = 5


def _sc_term1(src3, dst3, a3, emb):
    mesh = plsc.VectorSubcoreMesh(core_axis_name="c", subcore_axis_name="s")

    @functools.partial(
        pl.kernel,
        mesh=mesh,
        compiler_params=_sc_compiler_params(),
        out_type=jax.ShapeDtypeStruct((NW, LANES), jnp.float32),
        scratch_types=[
            pltpu.VMEM((NCHUNK, C), jnp.int32),
            pltpu.VMEM((NCHUNK, C), jnp.int32),
            pltpu.VMEM((NCHUNK, C), jnp.float32),
            pltpu.VMEM((C, K // 2), jnp.int32),
            pltpu.VMEM((C, K // 2), jnp.int32),
            pltpu.VMEM((C, K // 2), jnp.int32),
            pltpu.VMEM((C, K // 2), jnp.int32),
            pltpu.VMEM((LANES,), jnp.float32),
            pltpu.VMEM_SHARED((N, K // 2), jnp.int32),
            pltpu.SemaphoreType.DMA,
            pltpu.SemaphoreType.DMA,
            pltpu.SemaphoreType.DMA,
            pltpu.SemaphoreType.DMA,
        ],
    )
    def k(src_hbm, dst_hbm, a_hbm, emb_hbm, out_hbm,
          sidx_all, didx_all, av_all,
          sr0, dr0, sr1, dr1, accv, table,
          ss0, sd0, ss1, sd1):
        sid = lax.axis_index("s")
        wid = sid * NC + lax.axis_index("c")
        rows_per_tile = N // NS
        pltpu.sync_copy(emb_hbm.at[pl.ds(sid * rows_per_tile, rows_per_tile)],
                        table.at[pl.ds(sid * rows_per_tile, rows_per_tile)])
        pltpu.sync_copy(src_hbm.at[wid], sidx_all)
        pltpu.sync_copy(dst_hbm.at[wid], didx_all)
        pltpu.sync_copy(a_hbm.at[wid], av_all)
        accv[...] = jnp.zeros((LANES,), jnp.float32)
        plsc.subcore_barrier()

        bufs = ((sr0, dr0, ss0, sd0), (sr1, dr1, ss1, sd1))
        DEPTH = len(bufs)

        def issue(ci, sbuf, dbuf, ssem, dsem):
            pltpu.async_copy(table.at[sidx_all.at[ci]], sbuf, ssem)
            pltpu.async_copy(table.at[didx_all.at[ci]], dbuf, dsem)

        def wait(ci, sbuf, dbuf, ssem, dsem):
            pltpu.make_async_copy(table.at[sidx_all.at[ci]], sbuf, ssem).wait()
            pltpu.make_async_copy(table.at[didx_all.at[ci]], dbuf, dsem).wait()

        def compute(ci, sbuf, dbuf):
            def edge_group(g, acc):
                e0 = g * _UNROLL
                for u in range(_UNROLL):
                    e = e0 + u
                    t = jnp.zeros((LANES,), jnp.float32)
                    for j in range(K // (2 * LANES)):
                        sv = plsc.bitcast(
                            sbuf[e, pl.ds(j * LANES, LANES)], jnp.bfloat16)
                        dv = plsc.bitcast(
                            dbuf[e, pl.ds(j * LANES, LANES)], jnp.bfloat16)
                        p0, p1 = plsc.unpack(
                            sv * dv, format=plsc.PackFormat.INTERLEAVED)
                        t = t + p0 + p1
                    ab = plsc.load_gather(
                        av_all, [jnp.full((LANES,), ci, jnp.int32),
                                 jnp.full((LANES,), e, jnp.int32)])
                    acc = acc + ab * t
                return acc

            acc = lax.fori_loop(0, C // _UNROLL, edge_group,
                                jnp.zeros((LANES,), jnp.float32))
            accv[...] = accv[...] + acc

        for b in range(DEPTH):
            issue(b, *bufs[b])

        @pl.loop(0, NCHUNK, step=DEPTH)
        def _chunk(ci):
            for b in range(DEPTH):
                sbuf, dbuf, ssem, dsem = bufs[b]
                wait(ci + b, sbuf, dbuf, ssem, dsem)
                compute(ci + b, sbuf, dbuf)

                @pl.when(ci + b + DEPTH < NCHUNK)
                def _():
                    issue(ci + b + DEPTH, sbuf, dbuf, ssem, dsem)

        pltpu.sync_copy(accv, out_hbm.at[wid])

    return k(src3, dst3, a3, emb)


_BR = 1000


def _gram_body(e_ref, d_ref, ptp_ref, t2_ref):
    @pl.when(pl.program_id(0) == 0)
    def _():
        ptp_ref[...] = jnp.zeros((K, K), jnp.float32)
        t2_ref[0, 0] = 0.0

    blk = e_ref[...]
    ptp_ref[...] += lax.dot_general(blk, blk, (((0,), (0,)), ((), ())),
                                    preferred_element_type=jnp.float32)
    rs = jnp.sum(blk * blk, axis=1)
    t2_ref[0, 0] += jnp.sum(d_ref[...][:, 0] * rs)


def _tc_gram(emb, d_col):
    return pl.pallas_call(
        _gram_body,
        grid=(N // _BR,),
        in_specs=[
            pl.BlockSpec((_BR, K), lambda i: (i, 0)),
            pl.BlockSpec((_BR, 1), lambda i: (i, 0)),
        ],
        out_specs=[
            pl.BlockSpec((K, K), lambda i: (0, 0)),
            pl.BlockSpec(memory_space=pltpu.SMEM),
        ],
        out_shape=[
            jax.ShapeDtypeStruct((K, K), jnp.float32),
            jax.ShapeDtypeStruct((1, 1), jnp.float32),
        ],
    )(emb, d_col)


def _combine_body(ptp_ref, part_ref, t2_ref, out_ref):
    ptp = ptp_ref[...]
    term1 = jnp.sum(part_ref[...])
    term2 = t2_ref[0, 0]
    n = jnp.sqrt(jnp.sum(ptp * ptp))
    row = lax.broadcasted_iota(jnp.int32, (K, K), 0)
    col = lax.broadcasted_iota(jnp.int32, (K, K), 1)
    eye = jnp.where(row == col, jnp.float32(1.0), jnp.float32(0.0))
    m = ptp / n - eye / jnp.sqrt(jnp.float32(K))
    penalty = jnp.sqrt(jnp.sum(m * m))
    out_ref[0, 0] = -(term1 / term2) + penalty


def _tc_combine(ptp, partials, t2):
    return pl.pallas_call(
        _combine_body,
        in_specs=[
            pl.BlockSpec((K, K), lambda: (0, 0)),
            pl.BlockSpec((NW, LANES), lambda: (0, 0)),
            pl.BlockSpec(memory_space=pltpu.SMEM),
        ],
        out_specs=pl.BlockSpec(memory_space=pltpu.SMEM),
        out_shape=jax.ShapeDtypeStruct((1, 1), jnp.float32),
    )(ptp, partials, t2)


def kernel(embeddings, edge_index, edge_weight, normalized_A_values, D_values):
    del edge_weight
    src = edge_index[0].astype(jnp.int32).reshape(NW, NCHUNK, C)
    dst = edge_index[1].astype(jnp.int32).reshape(NW, NCHUNK, C)
    a3 = normalized_A_values.reshape(NW, NCHUNK, C)
    emb_packed = lax.bitcast_convert_type(
        embeddings.astype(jnp.bfloat16).reshape(N, K // 2, 2), jnp.int32)
    partials = _sc_term1(src, dst, a3, emb_packed)
    ptp, t2 = _tc_gram(embeddings, D_values.reshape(N, 1))
    out = _tc_combine(ptp, partials, t2)
    return out[0, 0]


# speedup vs baseline: 15.5013x; 1.3628x over previous
import dataclasses
import functools

import jax
import jax.numpy as jnp
from jax import lax
from jax.experimental import pallas as pl
from jax.experimental.pallas import tpu as pltpu
from jax.experimental.pallas import tpu_sc as plsc

N = 10000
E = 320000
K = 128

NC = 2
NS = 16
NW = NC * NS
EPW = E // NW
C = 80
NCHUNK = EPW // C
DEPTH = 5
LANES = 16
KP = K // 2


def _sc_compiler_params():
    cp = pltpu.CompilerParams()
    if "needs_layout_passes" in pltpu.CompilerParams.__dataclass_fields__:
        cp = dataclasses.replace(cp, needs_layout_passes=False)
    if "use_tc_tiling_on_sc" in pltpu.CompilerParams.__dataclass_fields__:
        cp = dataclasses.replace(cp, use_tc_tiling_on_sc=False)
    return cp


_UNROLL = 5


def _sc_term1(edge_index, a_vals, packed):
    mesh = plsc.VectorSubcoreMesh(core_axis_name="c", subcore_axis_name="s")

    @functools.partial(
        pl.kernel,
        mesh=mesh,
        compiler_params=_sc_compiler_params(),
        out_type=jax.ShapeDtypeStruct((NW, LANES), jnp.float32),
        scratch_types=[
            pltpu.VMEM((EPW,), jnp.int32),
            pltpu.VMEM((EPW,), jnp.int32),
            pltpu.VMEM((EPW,), jnp.float32),
            [pltpu.VMEM((C, KP), jnp.int32) for _ in range(2 * DEPTH)],
            pltpu.VMEM((LANES,), jnp.float32),
            pltpu.VMEM_SHARED((N, KP), jnp.int32),
            [pltpu.SemaphoreType.DMA for _ in range(2 * DEPTH)],
        ],
    )
    def k(ei_hbm, a_hbm, packed_hbm, out_hbm,
          sidx, didx, av, rowbufs, accv, table, sems):
        sid = lax.axis_index("s")
        wid = sid * NC + lax.axis_index("c")
        base = wid * EPW

        @pl.when(sid < 10)
        def _():
            pltpu.sync_copy(packed_hbm.at[pl.ds(sid * 1000, 1000)],
                            table.at[pl.ds(sid * 1000, 1000)])
        pltpu.sync_copy(ei_hbm.at[0, pl.ds(base, EPW)], sidx)
        pltpu.sync_copy(ei_hbm.at[1, pl.ds(base, EPW)], didx)
        pltpu.sync_copy(a_hbm.at[pl.ds(base, EPW)], av)
        accv[...] = jnp.zeros((LANES,), jnp.float32)
        plsc.subcore_barrier()

        bufs = tuple((rowbufs[2 * b], rowbufs[2 * b + 1],
                      sems[2 * b], sems[2 * b + 1]) for b in range(DEPTH))

        def issue(ci, sbuf, dbuf, ssem, dsem):
            pltpu.async_copy(table.at[sidx.at[pl.ds(ci * C, C)]], sbuf, ssem)
            pltpu.async_copy(table.at[didx.at[pl.ds(ci * C, C)]], dbuf, dsem)

        def wait(ci, sbuf, dbuf, ssem, dsem):
            pltpu.make_async_copy(
                table.at[sidx.at[pl.ds(ci * C, C)]], sbuf, ssem).wait()
            pltpu.make_async_copy(
                table.at[didx.at[pl.ds(ci * C, C)]], dbuf, dsem).wait()

        def compute(ci, sbuf, dbuf):
            def edge_group(g, acc):
                e0 = g * _UNROLL
                for u in range(_UNROLL):
                    e = e0 + u
                    t = jnp.zeros((LANES,), jnp.float32)
                    for j in range(KP // LANES):
                        sv = plsc.bitcast(
                            sbuf[e, pl.ds(j * LANES, LANES)], jnp.bfloat16)
                        dv = plsc.bitcast(
                            dbuf[e, pl.ds(j * LANES, LANES)], jnp.bfloat16)
                        p0, p1 = plsc.unpack(
                            sv * dv, format=plsc.PackFormat.INTERLEAVED)
                        t = t + p0 + p1
                    ab = plsc.load_gather(
                        av, [jnp.full((LANES,), ci * C + e, jnp.int32)])
                    acc = acc + ab * t
                return acc

            acc = lax.fori_loop(0, C // _UNROLL, edge_group,
                                jnp.zeros((LANES,), jnp.float32))
            accv[...] = accv[...] + acc

        for b in range(DEPTH):
            issue(b, *bufs[b])

        @pl.loop(0, NCHUNK, step=DEPTH)
        def _chunk(ci):
            for b in range(DEPTH):
                sbuf, dbuf, ssem, dsem = bufs[b]
                wait(ci + b, sbuf, dbuf, ssem, dsem)
                compute(ci + b, sbuf, dbuf)

                @pl.when(ci + b + DEPTH < NCHUNK)
                def _():
                    issue(ci + b + DEPTH, sbuf, dbuf, ssem, dsem)

        pltpu.sync_copy(accv, out_hbm.at[wid])

    return k(edge_index, a_vals, packed)


_BR = 1000


def _gram_body(e_ref, d_ref, ptp_ref, t2_ref, pk_ref):
    @pl.when(pl.program_id(0) == 0)
    def _():
        ptp_ref[...] = jnp.zeros((K, K), jnp.float32)
        t2_ref[0, 0] = 0.0

    blk = e_ref[...]
    ptp_ref[...] += lax.dot_general(blk, blk, (((0,), (0,)), ((), ())),
                                    preferred_element_type=jnp.float32)
    rs = jnp.sum(blk * blk, axis=1)
    t2_ref[0, 0] += jnp.sum(d_ref[...][:, 0] * rs)

    b = lax.bitcast_convert_type(blk, jnp.int32)
    r = lax.shift_right_logical(
        b + 0x7FFF + lax.shift_right_logical(b, 16) % 2, 16)
    pk_ref[...] = lax.shift_left(r[:, KP:], 16) | r[:, :KP]


def _tc_gram(emb, d_col):
    return pl.pallas_call(
        _gram_body,
        grid=(N // _BR,),
        in_specs=[
            pl.BlockSpec((_BR, K), lambda i: (i, 0)),
            pl.BlockSpec((_BR, 1), lambda i: (i, 0)),
        ],
        out_specs=[
            pl.BlockSpec((K, K), lambda i: (0, 0)),
            pl.BlockSpec(memory_space=pltpu.SMEM),
            pl.BlockSpec((_BR, KP), lambda i: (i, 0)),
        ],
        out_shape=[
            jax.ShapeDtypeStruct((K, K), jnp.float32),
            jax.ShapeDtypeStruct((1, 1), jnp.float32),
            jax.ShapeDtypeStruct((N, KP), jnp.int32),
        ],
    )(emb, d_col)


def _combine_body(ptp_ref, part_ref, t2_ref, out_ref):
    ptp = ptp_ref[...]
    term1 = jnp.sum(part_ref[...])
    term2 = t2_ref[0, 0]
    n = jnp.sqrt(jnp.sum(ptp * ptp))
    row = lax.broadcasted_iota(jnp.int32, (K, K), 0)
    col = lax.broadcasted_iota(jnp.int32, (K, K), 1)
    eye = jnp.where(row == col, jnp.float32(1.0), jnp.float32(0.0))
    m = ptp / n - eye / jnp.sqrt(jnp.float32(K))
    penalty = jnp.sqrt(jnp.sum(m * m))
    out_ref[0, 0] = -(term1 / term2) + penalty


def _tc_combine(ptp, partials, t2):
    return pl.pallas_call(
        _combine_body,
        in_specs=[
            pl.BlockSpec((K, K), lambda: (0, 0)),
            pl.BlockSpec((NW, LANES), lambda: (0, 0)),
            pl.BlockSpec(memory_space=pltpu.SMEM),
        ],
        out_specs=pl.BlockSpec(memory_space=pltpu.SMEM),
        out_shape=jax.ShapeDtypeStruct((1, 1), jnp.float32),
    )(ptp, partials, t2)


def kernel(embeddings, edge_index, edge_weight, normalized_A_values, D_values):
    del edge_weight
    ptp, t2, packed = _tc_gram(embeddings, D_values.reshape(N, 1))
    partials = _sc_term1(edge_index.astype(jnp.int32), normalized_A_values,
                         packed)
    out = _tc_combine(ptp, partials, t2)
    return out[0, 0]


# speedup vs baseline: 16.1830x; 1.0440x over previous
import dataclasses
import functools

import jax
import jax.numpy as jnp
from jax import lax
from jax.experimental import pallas as pl
from jax.experimental.pallas import tpu as pltpu
from jax.experimental.pallas import tpu_sc as plsc

N = 10000
E = 320000
K = 128

NC = 2
NS = 16
NW = NC * NS
EPW = E // NW
C = 80
NCHUNK = EPW // C
DEPTH = 5
LANES = 16
KP = K // 2


def _sc_compiler_params():
    cp = pltpu.CompilerParams()
    if "needs_layout_passes" in pltpu.CompilerParams.__dataclass_fields__:
        cp = dataclasses.replace(cp, needs_layout_passes=False)
    if "use_tc_tiling_on_sc" in pltpu.CompilerParams.__dataclass_fields__:
        cp = dataclasses.replace(cp, use_tc_tiling_on_sc=False)
    return cp


_UNROLL = 5


def _sc_term1(edge_index, a_vals, packed):
    mesh = plsc.VectorSubcoreMesh(core_axis_name="c", subcore_axis_name="s")

    @functools.partial(
        pl.kernel,
        mesh=mesh,
        compiler_params=_sc_compiler_params(),
        out_type=jax.ShapeDtypeStruct((8, 128), jnp.float32),
        scratch_types=[
            pltpu.VMEM((EPW,), jnp.int32),
            pltpu.VMEM((EPW,), jnp.int32),
            pltpu.VMEM((EPW,), jnp.float32),
            [pltpu.VMEM((C, KP), jnp.int32) for _ in range(2 * DEPTH)],
            pltpu.VMEM((LANES,), jnp.float32),
            pltpu.VMEM_SHARED((N, KP), jnp.int32),
            [pltpu.SemaphoreType.DMA for _ in range(2 * DEPTH)],
        ],
    )
    def k(ei_hbm, a_hbm, packed_hbm, out_hbm,
          sidx, didx, av, rowbufs, accv, table, sems):
        sid = lax.axis_index("s")
        wid = sid * NC + lax.axis_index("c")
        base = wid * EPW

        @pl.when(sid < 10)
        def _():
            pltpu.sync_copy(packed_hbm.at[pl.ds(sid * 1000, 1000)],
                            table.at[pl.ds(sid * 1000, 1000)])
        pltpu.sync_copy(ei_hbm.at[0, pl.ds(base, EPW)], sidx)
        pltpu.sync_copy(ei_hbm.at[1, pl.ds(base, EPW)], didx)
        pltpu.sync_copy(a_hbm.at[pl.ds(base, EPW)], av)
        accv[...] = jnp.zeros((LANES,), jnp.float32)
        plsc.subcore_barrier()

        bufs = tuple((rowbufs[2 * b], rowbufs[2 * b + 1],
                      sems[2 * b], sems[2 * b + 1]) for b in range(DEPTH))

        def issue(ci, sbuf, dbuf, ssem, dsem):
            pltpu.async_copy(table.at[sidx.at[pl.ds(ci * C, C)]], sbuf, ssem)
            pltpu.async_copy(table.at[didx.at[pl.ds(ci * C, C)]], dbuf, dsem)

        def wait(ci, sbuf, dbuf, ssem, dsem):
            pltpu.make_async_copy(
                table.at[sidx.at[pl.ds(ci * C, C)]], sbuf, ssem).wait()
            pltpu.make_async_copy(
                table.at[didx.at[pl.ds(ci * C, C)]], dbuf, dsem).wait()

        def compute(ci, sbuf, dbuf):
            def edge_group(g, acc):
                e0 = g * _UNROLL
                for u in range(_UNROLL):
                    e = e0 + u
                    t = jnp.zeros((LANES,), jnp.float32)
                    for j in range(KP // LANES):
                        sv = plsc.bitcast(
                            sbuf[e, pl.ds(j * LANES, LANES)], jnp.bfloat16)
                        dv = plsc.bitcast(
                            dbuf[e, pl.ds(j * LANES, LANES)], jnp.bfloat16)
                        p0, p1 = plsc.unpack(
                            sv * dv, format=plsc.PackFormat.INTERLEAVED)
                        t = t + p0 + p1
                    ab = plsc.load_gather(
                        av, [jnp.full((LANES,), ci * C + e, jnp.int32)])
                    acc = acc + ab * t
                return acc

            acc = lax.fori_loop(0, C // _UNROLL, edge_group,
                                jnp.zeros((LANES,), jnp.float32))
            accv[...] = accv[...] + acc

        for b in range(DEPTH):
            issue(b, *bufs[b])

        @pl.loop(0, NCHUNK, step=DEPTH)
        def _chunk(ci):
            for b in range(DEPTH):
                sbuf, dbuf, ssem, dsem = bufs[b]
                wait(ci + b, sbuf, dbuf, ssem, dsem)
                compute(ci + b, sbuf, dbuf)

                @pl.when(ci + b + DEPTH < NCHUNK)
                def _():
                    issue(ci + b + DEPTH, sbuf, dbuf, ssem, dsem)

        pltpu.sync_copy(accv,
                        out_hbm.at[wid % 8, pl.ds((wid // 8) * LANES, LANES)])

    return k(edge_index, a_vals, packed)


_BR = 1000


def _gram_body(e_ref, d_ref, ptp_ref, t2_ref, pk_ref):
    @pl.when(pl.program_id(0) == 0)
    def _():
        ptp_ref[...] = jnp.zeros((K, K), jnp.float32)
        t2_ref[0, 0] = 0.0

    blk = e_ref[...]
    blk16 = blk.astype(jnp.bfloat16)
    ptp_ref[...] += lax.dot_general(blk16, blk16, (((0,), (0,)), ((), ())),
                                    preferred_element_type=jnp.float32)
    rs = jnp.sum(blk * blk, axis=1)
    drow = d_ref[pl.ds(pl.program_id(0), 1), :]
    t2_ref[0, 0] += jnp.sum(drow[0, :] * rs)

    b = lax.bitcast_convert_type(blk, jnp.int32)
    r = lax.shift_right_logical(
        b + 0x7FFF + lax.shift_right_logical(b, 16) % 2, 16)
    pk_ref[...] = lax.shift_left(r[:, KP:], 16) | r[:, :KP]


def _tc_gram(emb, d_col):
    return pl.pallas_call(
        _gram_body,
        grid=(N // _BR,),
        in_specs=[
            pl.BlockSpec((_BR, K), lambda i: (i, 0)),
            pl.BlockSpec((N // _BR, _BR), lambda i: (0, 0)),
        ],
        out_specs=[
            pl.BlockSpec((K, K), lambda i: (0, 0)),
            pl.BlockSpec(memory_space=pltpu.SMEM),
            pl.BlockSpec((_BR, KP), lambda i: (i, 0)),
        ],
        out_shape=[
            jax.ShapeDtypeStruct((K, K), jnp.float32),
            jax.ShapeDtypeStruct((1, 1), jnp.float32),
            jax.ShapeDtypeStruct((N, KP), jnp.int32),
        ],
    )(emb, d_col)


def _combine_body(ptp_ref, part_ref, t2_ref, out_ref):
    ptp = ptp_ref[...]
    term1 = jnp.sum(part_ref[...][:, :NW // 8 * LANES])
    term2 = t2_ref[0, 0]
    n = jnp.sqrt(jnp.sum(ptp * ptp))
    row = lax.broadcasted_iota(jnp.int32, (K, K), 0)
    col = lax.broadcasted_iota(jnp.int32, (K, K), 1)
    eye = jnp.where(row == col, jnp.float32(1.0), jnp.float32(0.0))
    m = ptp / n - eye / jnp.sqrt(jnp.float32(K))
    penalty = jnp.sqrt(jnp.sum(m * m))
    out_ref[0, 0] = -(term1 / term2) + penalty


def _tc_combine(ptp, partials, t2):
    return pl.pallas_call(
        _combine_body,
        in_specs=[
            pl.BlockSpec((K, K), lambda: (0, 0)),
            pl.BlockSpec((8, 128), lambda: (0, 0)),
            pl.BlockSpec(memory_space=pltpu.SMEM),
        ],
        out_specs=pl.BlockSpec(memory_space=pltpu.SMEM),
        out_shape=jax.ShapeDtypeStruct((1, 1), jnp.float32),
    )(ptp, partials, t2)


def kernel(embeddings, edge_index, edge_weight, normalized_A_values, D_values):
    del edge_weight
    ptp, t2, packed = _tc_gram(embeddings, D_values.reshape(N // _BR, _BR))
    partials = _sc_term1(edge_index.astype(jnp.int32), normalized_A_values,
                         packed)
    out = _tc_combine(ptp, partials, t2)
    return out[0, 0]


# speedup vs baseline: 16.2151x; 1.0020x over previous
import dataclasses
import functools

import jax
import jax.numpy as jnp
from jax import lax
from jax.experimental import pallas as pl
from jax.experimental.pallas import tpu as pltpu
from jax.experimental.pallas import tpu_sc as plsc

N = 10000
E = 320000
K = 128

NC = 2
NS = 16
NW = NC * NS
EPW = E // NW
C = 80
NCHUNK = EPW // C
DEPTH = 5
LANES = 16
KP = K // 2


def _sc_compiler_params():
    cp = pltpu.CompilerParams()
    if "needs_layout_passes" in pltpu.CompilerParams.__dataclass_fields__:
        cp = dataclasses.replace(cp, needs_layout_passes=False)
    if "use_tc_tiling_on_sc" in pltpu.CompilerParams.__dataclass_fields__:
        cp = dataclasses.replace(cp, use_tc_tiling_on_sc=False)
    return cp


_UNROLL = 5


def _sc_term1(edge_index, a_vals, packed):
    mesh = plsc.VectorSubcoreMesh(core_axis_name="c", subcore_axis_name="s")

    @functools.partial(
        pl.kernel,
        mesh=mesh,
        compiler_params=_sc_compiler_params(),
        out_type=jax.ShapeDtypeStruct((8, 128), jnp.float32),
        scratch_types=[
            pltpu.VMEM((EPW,), jnp.int32),
            pltpu.VMEM((EPW,), jnp.int32),
            pltpu.VMEM((EPW,), jnp.float32),
            [pltpu.VMEM((C, KP), jnp.int32) for _ in range(2 * DEPTH)],
            pltpu.VMEM((LANES,), jnp.float32),
            pltpu.VMEM_SHARED((N, KP), jnp.int32),
            [pltpu.SemaphoreType.DMA for _ in range(2 * DEPTH)],
        ],
    )
    def k(ei_hbm, a_hbm, packed_hbm, out_hbm,
          sidx, didx, av, rowbufs, accv, table, sems):
        sid = lax.axis_index("s")
        wid = sid * NC + lax.axis_index("c")
        base = wid * EPW

        @pl.when(sid < 10)
        def _():
            pltpu.sync_copy(packed_hbm.at[pl.ds(sid * 1000, 1000)],
                            table.at[pl.ds(sid * 1000, 1000)])
        pltpu.sync_copy(ei_hbm.at[0, pl.ds(base, EPW)], sidx)
        pltpu.sync_copy(ei_hbm.at[1, pl.ds(base, EPW)], didx)
        pltpu.sync_copy(a_hbm.at[pl.ds(base, EPW)], av)
        accv[...] = jnp.zeros((LANES,), jnp.float32)
        plsc.subcore_barrier()

        bufs = tuple((rowbufs[2 * b], rowbufs[2 * b + 1],
                      sems[2 * b], sems[2 * b + 1]) for b in range(DEPTH))

        def issue(ci, sbuf, dbuf, ssem, dsem):
            pltpu.async_copy(table.at[sidx.at[pl.ds(ci * C, C)]], sbuf, ssem)
            pltpu.async_copy(table.at[didx.at[pl.ds(ci * C, C)]], dbuf, dsem)

        def wait(ci, sbuf, dbuf, ssem, dsem):
            pltpu.make_async_copy(
                table.at[sidx.at[pl.ds(ci * C, C)]], sbuf, ssem).wait()
            pltpu.make_async_copy(
                table.at[didx.at[pl.ds(ci * C, C)]], dbuf, dsem).wait()

        def compute(ci, sbuf, dbuf):
            def edge_group(g, acc):
                e0 = g * _UNROLL
                for u in range(_UNROLL):
                    e = e0 + u
                    t = jnp.zeros((LANES,), jnp.float32)
                    for j in range(KP // LANES):
                        sv = plsc.bitcast(
                            sbuf[e, pl.ds(j * LANES, LANES)], jnp.bfloat16)
                        dv = plsc.bitcast(
                            dbuf[e, pl.ds(j * LANES, LANES)], jnp.bfloat16)
                        p0, p1 = plsc.unpack(
                            sv * dv, format=plsc.PackFormat.INTERLEAVED)
                        t = t + p0 + p1
                    ab = plsc.load_gather(
                        av, [jnp.full((LANES,), ci * C + e, jnp.int32)])
                    acc = acc + ab * t
                return acc

            acc = lax.fori_loop(0, C // _UNROLL, edge_group,
                                jnp.zeros((LANES,), jnp.float32))
            accv[...] = accv[...] + acc

        for b in range(DEPTH):
            issue(b, *bufs[b])

        @pl.loop(0, NCHUNK, step=DEPTH)
        def _chunk(ci):
            for b in range(DEPTH):
                sbuf, dbuf, ssem, dsem = bufs[b]
                wait(ci + b, sbuf, dbuf, ssem, dsem)
                compute(ci + b, sbuf, dbuf)

                @pl.when(ci + b + DEPTH < NCHUNK)
                def _():
                    issue(ci + b + DEPTH, sbuf, dbuf, ssem, dsem)

        pltpu.sync_copy(accv,
                        out_hbm.at[wid % 8, pl.ds((wid // 8) * LANES, LANES)])

    return k(edge_index, a_vals, packed)


_BR = 1000


def _gram_body(e_ref, d_ref, ptp_ref, t2_ref, pk_ref):
    @pl.when(pl.program_id(0) == 0)
    def _():
        ptp_ref[...] = jnp.zeros((K, K), jnp.float32)
        t2_ref[0, 0] = 0.0

    blk = e_ref[...]
    ptp_ref[...] += lax.dot_general(blk, blk, (((0,), (0,)), ((), ())),
                                    preferred_element_type=jnp.float32)
    rs = jnp.sum(blk * blk, axis=1)
    drow = d_ref[pl.ds(pl.program_id(0), 1), :]
    t2_ref[0, 0] += jnp.sum(drow[0, :] * rs)

    b = lax.bitcast_convert_type(blk, jnp.int32)
    r = lax.shift_right_logical(
        b + 0x7FFF + lax.shift_right_logical(b, 16) % 2, 16)
    pk_ref[...] = lax.shift_left(r[:, KP:], 16) | r[:, :KP]


def _tc_gram(emb, d_col):
    return pl.pallas_call(
        _gram_body,
        grid=(N // _BR,),
        in_specs=[
            pl.BlockSpec((_BR, K), lambda i: (i, 0)),
            pl.BlockSpec((N // _BR, _BR), lambda i: (0, 0)),
        ],
        out_specs=[
            pl.BlockSpec((K, K), lambda i: (0, 0)),
            pl.BlockSpec(memory_space=pltpu.SMEM),
            pl.BlockSpec((_BR, KP), lambda i: (i, 0)),
        ],
        out_shape=[
            jax.ShapeDtypeStruct((K, K), jnp.float32),
            jax.ShapeDtypeStruct((1, 1), jnp.float32),
            jax.ShapeDtypeStruct((N, KP), jnp.int32),
        ],
    )(emb, d_col)


def _combine_body(ptp_ref, part_ref, t2_ref, out_ref):
    ptp = ptp_ref[...]
    term1 = jnp.sum(part_ref[...][:, :NW // 8 * LANES])
    term2 = t2_ref[0, 0]
    n = jnp.sqrt(jnp.sum(ptp * ptp))
    row = lax.broadcasted_iota(jnp.int32, (K, K), 0)
    col = lax.broadcasted_iota(jnp.int32, (K, K), 1)
    eye = jnp.where(row == col, jnp.float32(1.0), jnp.float32(0.0))
    m = ptp / n - eye / jnp.sqrt(jnp.float32(K))
    penalty = jnp.sqrt(jnp.sum(m * m))
    out_ref[0, 0] = -(term1 / term2) + penalty


def _tc_combine(ptp, partials, t2):
    return pl.pallas_call(
        _combine_body,
        in_specs=[
            pl.BlockSpec((K, K), lambda: (0, 0)),
            pl.BlockSpec((8, 128), lambda: (0, 0)),
            pl.BlockSpec(memory_space=pltpu.SMEM),
        ],
        out_specs=pl.BlockSpec(memory_space=pltpu.SMEM),
        out_shape=jax.ShapeDtypeStruct((1, 1), jnp.float32),
    )(ptp, partials, t2)


def kernel(embeddings, edge_index, edge_weight, normalized_A_values, D_values):
    del edge_weight
    ptp, t2, packed = _tc_gram(embeddings, D_values.reshape(N // _BR, _BR))
    partials = _sc_term1(edge_index.astype(jnp.int32), normalized_A_values,
                         packed)
    out = _tc_combine(ptp, partials, t2)
    return out[0, 0]
